# Initial kernel scaffold; baseline (speedup 1.0000x reference)
import jax
import jax.numpy as jnp
from jax.experimental import pallas as pl


def kernel(x, edge_index, W1, W2, W3, W4, W5, W6, b1, b2, b3, b4, b5, b6):
    raise NotImplementedError("write your pallas kernel here")


# speedup vs baseline: 9.1727x; 9.1727x over previous
import functools

import jax
import jax.numpy as jnp
from jax import lax
from jax.experimental import pallas as pl
from jax.experimental.pallas import tpu as pltpu
from jax.experimental.pallas import tpu_sc as plsc

N = 10000
D = 128
E = 320000

NC = 2
NS = 16
NW = NC * NS
EW = 80
EPW = E // NW
WIN_PER_WORKER = EPW // EW
RBLK = 80
NRB = N // RBLK

_MESH = plsc.VectorSubcoreMesh(core_axis_name="c", subcore_axis_name="s")

RB = 2000
GRID = N // RB


def _sc_degree(dst1):
    @functools.partial(
        pl.kernel,
        out_type=jax.ShapeDtypeStruct((NC, N, 16), jnp.float32),
        mesh=_MESH,
        scratch_types=[
            pltpu.VMEM((1, EW), jnp.int32),
            pltpu.VMEM((EW, 16), jnp.float32),
            pltpu.VMEM((RBLK, 16), jnp.float32),
            pltpu.VMEM_SHARED((N, 16), jnp.float32),
        ],
    )
    def k(dst_hbm, out_hbm, didx, ones, zbuf, acc):
        cid = lax.axis_index("c")
        sid = lax.axis_index("s")
        wid = sid * NC + cid

        @pl.loop(0, EW)
        def _(i):
            ones[i] = jnp.ones((16,), jnp.float32)

        @pl.loop(0, RBLK)
        def _(i):
            zbuf[i] = jnp.zeros((16,), jnp.float32)

        @pl.loop(sid, NRB, step=NS)
        def _(blk):
            pltpu.sync_copy(zbuf, acc.at[pl.ds(blk * RBLK, RBLK)])

        plsc.subcore_barrier()

        @pl.loop(0, WIN_PER_WORKER)
        def _(w):
            off = wid * EPW + w * EW
            pltpu.sync_copy(dst_hbm.at[pl.ds(off, EW)], didx.at[0])
            pltpu.sync_copy(ones, acc.at[didx.at[0]], add=True)

        plsc.subcore_barrier()

        @pl.loop(sid, NRB, step=NS)
        def _(blk):
            r0 = blk * RBLK
            pltpu.sync_copy(acc.at[pl.ds(r0, RBLK)],
                            out_hbm.at[cid].at[pl.ds(r0, RBLK)])

    return k(dst1)


def _sc_edge_pass(y, src1, dst1):
    @functools.partial(
        pl.kernel,
        out_type=jax.ShapeDtypeStruct((NC, N, D), jnp.float32),
        mesh=_MESH,
        scratch_types=[
            pltpu.VMEM((1, EW), jnp.int32),
            pltpu.VMEM((1, EW), jnp.int32),
            pltpu.VMEM((EW, D), jnp.float32),
            pltpu.VMEM_SHARED((N, D), jnp.float32),
            pltpu.SemaphoreType.DMA,
        ],
    )
    def k(y_hbm, src_hbm, dst_hbm, out_hbm, sidx, didx, rows, acc, sem):
        cid = lax.axis_index("c")
        sid = lax.axis_index("s")
        wid = sid * NC + cid

        @pl.loop(sid, NRB, step=NS)
        def _(blk):
            r0 = blk * RBLK
            pltpu.sync_copy(y_hbm.at[pl.ds(r0, RBLK)],
                            acc.at[pl.ds(r0, RBLK)])

        plsc.subcore_barrier()

        @pl.loop(0, WIN_PER_WORKER)
        def _(w):
            off = wid * EPW + w * EW
            pltpu.sync_copy(src_hbm.at[pl.ds(off, EW)], sidx.at[0])
            pltpu.sync_copy(dst_hbm.at[pl.ds(off, EW)], didx.at[0])
            pltpu.async_copy(y_hbm.at[sidx.at[0]], rows, sem).wait()
            pltpu.sync_copy(rows, acc.at[didx.at[0]], add=True)

        plsc.subcore_barrier()

        @pl.loop(sid, NRB, step=NS)
        def _(blk):
            r0 = blk * RBLK
            pltpu.sync_copy(acc.at[pl.ds(r0, RBLK)],
                            out_hbm.at[cid].at[pl.ds(r0, RBLK)])

    return k(y, src1, dst1)


def _prep_body(cnt_ref, x_ref, w_ref, y_ref, dinv_ref):
    deg = cnt_ref[0, :, 0:1] + cnt_ref[1, :, 0:1] + 1.0
    dinv = lax.rsqrt(deg)
    dinv_ref[...] = dinv
    y_ref[...] = dinv * jnp.dot(x_ref[...], w_ref[...],
                                preferred_element_type=jnp.float32)


def _tc_prep(cnt, x, w1):
    return pl.pallas_call(
        _prep_body,
        grid=(GRID,),
        in_specs=[
            pl.BlockSpec((NC, RB, 16), lambda i: (0, i, 0)),
            pl.BlockSpec((RB, D), lambda i: (i, 0)),
            pl.BlockSpec((D, D), lambda i: (0, 0)),
        ],
        out_specs=[
            pl.BlockSpec((RB, D), lambda i: (i, 0)),
            pl.BlockSpec((RB, 1), lambda i: (i, 0)),
        ],
        out_shape=[
            jax.ShapeDtypeStruct((N, D), jnp.float32),
            jax.ShapeDtypeStruct((N, 1), jnp.float32),
        ],
    )(cnt, x, w1)


def _mid_body(g_ref, y_ref, dinv_ref, b_ref, w_ref, out_ref):
    dinv = dinv_ref[...]
    h = g_ref[0] + g_ref[1] - y_ref[...]
    h = jnp.maximum(dinv * h + b_ref[...], 0.0)
    out_ref[...] = dinv * jnp.dot(h, w_ref[...],
                                  preferred_element_type=jnp.float32)


def _tc_mid(g, y, dinv, b, w):
    return pl.pallas_call(
        _mid_body,
        grid=(GRID,),
        in_specs=[
            pl.BlockSpec((NC, RB, D), lambda i: (0, i, 0)),
            pl.BlockSpec((RB, D), lambda i: (i, 0)),
            pl.BlockSpec((RB, 1), lambda i: (i, 0)),
            pl.BlockSpec((1, D), lambda i: (0, 0)),
            pl.BlockSpec((D, D), lambda i: (0, 0)),
        ],
        out_specs=pl.BlockSpec((RB, D), lambda i: (i, 0)),
        out_shape=jax.ShapeDtypeStruct((N, D), jnp.float32),
    )(g, y, dinv, b, w)


def _post_body(g_ref, y_ref, dinv_ref, b_ref, out_ref):
    h = g_ref[0] + g_ref[1] - y_ref[...]
    h = jnp.maximum(dinv_ref[...] * h + b_ref[...], 0.0)
    m = jnp.max(h, axis=-1, keepdims=True)
    s = h - m
    out_ref[...] = s - jnp.log(jnp.sum(jnp.exp(s), axis=-1, keepdims=True))


def _tc_post(g, y, dinv, b):
    return pl.pallas_call(
        _post_body,
        grid=(GRID,),
        in_specs=[
            pl.BlockSpec((NC, RB, D), lambda i: (0, i, 0)),
            pl.BlockSpec((RB, D), lambda i: (i, 0)),
            pl.BlockSpec((RB, 1), lambda i: (i, 0)),
            pl.BlockSpec((1, D), lambda i: (0, 0)),
        ],
        out_specs=pl.BlockSpec((RB, D), lambda i: (i, 0)),
        out_shape=jax.ShapeDtypeStruct((N, D), jnp.float32),
    )(g, y, dinv, b)


def kernel(x, edge_index, W1, W2, W3, W4, W5, W6, b1, b2, b3, b4, b5, b6):
    src1 = edge_index[0]
    dst1 = edge_index[1]
    Ws = (W1, W2, W3, W4, W5, W6)
    bs = (b1.reshape(1, D), b2.reshape(1, D), b3.reshape(1, D),
          b4.reshape(1, D), b5.reshape(1, D), b6.reshape(1, D))

    cnt = _sc_degree(dst1)
    y, dinv = _tc_prep(cnt, x, Ws[0])
    for i in range(5):
        g = _sc_edge_pass(y, src1, dst1)
        y = _tc_mid(g, y, dinv, bs[i], Ws[i + 1])
    g = _sc_edge_pass(y, src1, dst1)
    return _tc_post(g, y, dinv, bs[5])
